# Initial kernel scaffold; baseline (speedup 1.0000x reference)
import jax
import jax.numpy as jnp
from jax.experimental import pallas as pl


def kernel(xyz, features, offset, velocities, Wq, bq, Wk, bk, Wv, bv, Wp1, bp1, gp, betp, Wp2, bp2, Wvel1, bvel1, gvel, betvel, Wvel2, bvel2, gw1, bw1g, Ww1, bw1, gw2, bw2g, Ww2, bw2, gr, brg, Wr, br):
    raise NotImplementedError("write your pallas kernel here")


# speedup vs baseline: 2.2645x; 2.2645x over previous
import functools

import jax
import jax.numpy as jnp
from jax.experimental import pallas as pl
from jax.experimental.pallas import tpu as pltpu

_BIG = 3.0e38
_NSL = 16
_R = 256
_P = 1000


def _knn_body(q_ref, rT_ref, ssq_ref, ssr_ref, out_ref, *, n_chunks, payload):
    lane128 = jax.lax.broadcasted_iota(jnp.int32, (8, 128), 1)
    lane512 = jax.lax.broadcasted_iota(jnp.int32, (8, 512), 1).astype(jnp.float32)
    rTb = rT_ref[...].astype(jnp.bfloat16)

    def slab(r, carry):
        r8 = pl.multiple_of(r * 8, 8)
        q = q_ref[pl.ds(r8, 8), :]
        qq = ssq_ref[pl.ds(r8, 8), :]
        if payload:
            mm = q * rT_ref[...]
        else:
            mm = jnp.dot(q.astype(jnp.bfloat16), rTb,
                         preferred_element_type=jnp.float32)

        def chunk(c, bufs):
            b0, b1, b2, b3, p0, p1, p2, p3 = bufs
            base = c * 128
            d2 = (qq + ssr_ref[0:1, base:base + 128]) \
                - 2.0 * mm[:, base:base + 128]
            if payload:
                pay = jnp.broadcast_to(rT_ref[0:1, base:base + 128], (8, 128))
            else:
                pay = lane128 + c * 128
            m3 = d2 < b3
            nv = jnp.where(m3, d2, b3)
            np_ = jnp.where(m3, pay, p3)
            m2 = nv < b2
            b3n = jnp.where(m2, b2, nv)
            p3n = jnp.where(m2, p2, np_)
            nv2 = jnp.where(m2, nv, b2)
            np2 = jnp.where(m2, np_, p2)
            m1 = nv2 < b1
            b2n = jnp.where(m1, b1, nv2)
            p2n = jnp.where(m1, p1, np2)
            nv1 = jnp.where(m1, nv2, b1)
            np1 = jnp.where(m1, np2, p1)
            m0 = nv1 < b0
            b1n = jnp.where(m0, b0, nv1)
            p1n = jnp.where(m0, p0, np1)
            b0n = jnp.where(m0, nv1, b0)
            p0n = jnp.where(m0, np1, p0)
            return b0n, b1n, b2n, b3n, p0n, p1n, p2n, p3n

        fb = jnp.full((8, 128), _BIG, jnp.float32)
        if payload:
            fp = jnp.zeros((8, 128), jnp.float32)
        else:
            fp = jnp.zeros((8, 128), jnp.int32)
        bufs = (fb, fb, fb, fb, fp, fp, fp, fp)
        for c in range(n_chunks):
            bufs = chunk(c, bufs)
        b0, b1, b2, b3, p0, p1, p2, p3 = bufs
        B = jnp.concatenate([b0, b1, b2, b3], axis=1)
        P = jnp.concatenate([p0, p1, p2, p3], axis=1)
        outs = []
        for _ in range(_NSL):
            rowmin = jnp.min(B, axis=1, keepdims=True)
            eq = B == rowmin
            pos = jnp.min(jnp.where(eq, lane512, 1e9), axis=1, keepdims=True)
            sel = lane512 == pos
            if payload:
                val = jnp.sum(jnp.where(sel, P, 0.0), axis=1, keepdims=True)
            else:
                val = jnp.sum(jnp.where(sel, P, 0), axis=1, keepdims=True)
            outs.append(val)
            B = jnp.where(sel, _BIG, B)
        out_ref[pl.ds(r8, 8), :] = jnp.concatenate(outs, axis=1)
        return carry

    jax.lax.fori_loop(0, _R // 8, slab, 0)


def _knn(coords, payload):
    n, d = coords.shape
    npad = -(-n // _R) * _R
    pad = jnp.full((npad - n, d), 1e6, jnp.float32)
    cpad = jnp.concatenate([coords.astype(jnp.float32), pad], axis=0)
    cT = cpad.T
    ss = jnp.sum(cpad * cpad, axis=1)
    out_dtype = jnp.float32 if payload else jnp.int32
    body = functools.partial(_knn_body, n_chunks=npad // 128, payload=payload)
    return pl.pallas_call(
        body,
        grid=(npad // _R,),
        in_specs=[
            pl.BlockSpec((_R, d), lambda i: (i, 0)),
            pl.BlockSpec((d, npad), lambda i: (0, 0)),
            pl.BlockSpec((_R, 1), lambda i: (i, 0)),
            pl.BlockSpec((1, npad), lambda i: (0, 0)),
        ],
        out_specs=pl.BlockSpec((_R, _NSL), lambda i: (i, 0)),
        out_shape=jax.ShapeDtypeStruct((npad, _NSL), out_dtype),
        compiler_params=pltpu.CompilerParams(
            dimension_semantics=("arbitrary",)),
    )(cpad, cT, ss[:, None], ss[None, :])


def _passA_body(g_ref, xyz_ref, vg_ref, out_ref):
    first = (pl.program_id(0) == 0) & (pl.program_id(1) == 0)

    @pl.when(first)
    def _():
        out_ref[...] = jnp.zeros_like(out_ref)

    g = g_ref[0]
    pr = g[:, 64:67] - xyz_ref[...]
    v = vg_ref[0]
    s3 = jnp.sum(pr, axis=0, keepdims=True)
    sq = jnp.sum(pr * pr, axis=0, keepdims=True)
    sxy = jnp.sum(pr[:, 0:1] * pr[:, 1:2], axis=0, keepdims=True)
    sxz = jnp.sum(pr[:, 0:1] * pr[:, 2:3], axis=0, keepdims=True)
    syz = jnp.sum(pr[:, 1:2] * pr[:, 2:3], axis=0, keepdims=True)
    sv = jnp.sum(v, axis=0, keepdims=True)
    svv = jnp.sum(v * v, axis=0, keepdims=True)
    row = jnp.concatenate(
        [s3, sq, sxy, sxz, syz, sv, svv, jnp.zeros((1, 5), jnp.float32)], axis=1)
    out_ref[...] += row


def _passA(g3, xyz, vg3, nb, p):
    return pl.pallas_call(
        _passA_body,
        grid=(_NSL, nb),
        in_specs=[
            pl.BlockSpec((1, p, 80), lambda s, i: (s, i, 0)),
            pl.BlockSpec((p, 3), lambda s, i: (i, 0)),
            pl.BlockSpec((1, p, 1), lambda s, i: (s, i, 0)),
        ],
        out_specs=pl.BlockSpec((1, 16), lambda s, i: (0, 0)),
        out_shape=jax.ShapeDtypeStruct((1, 16), jnp.float32),
        compiler_params=pltpu.CompilerParams(
            dimension_semantics=("arbitrary", "arbitrary")),
    )(g3, xyz, vg3)


def _enc_terms(g2, xyz_blk, v_blk, wp1f_ref, bp1f_ref, wp2T_ref, bp2_ref,
               velc_ref, wvel2T_ref, bvel2_ref):
    pr = g2[:, 64:67] - xyz_blk
    pe = jax.nn.relu(
        jnp.dot(pr, wp1f_ref[...], preferred_element_type=jnp.float32)
        + bp1f_ref[...])
    p_enc = jnp.dot(pe, wp2T_ref[...], preferred_element_type=jnp.float32) \
        + bp2_ref[...]
    av = velc_ref[0:1, 0:1]
    bv = velc_ref[0:1, 1:2]
    a = jax.nn.relu(v_blk * av + bv)
    v_enc = a * wvel2T_ref[...] + bvel2_ref[...]
    return p_enc + v_enc


def _passB_body(g_ref, f_ref, xyz_ref, vg_ref, wqT_ref, bq_ref, wkT_ref,
                bk_ref, wp1f_ref, bp1f_ref, wp2T_ref, bp2_ref, velc_ref,
                wvel2T_ref, bvel2_ref, out_ref):
    first = (pl.program_id(0) == 0) & (pl.program_id(1) == 0)

    @pl.when(first)
    def _():
        out_ref[...] = jnp.zeros_like(out_ref)

    g = g_ref[0]
    enc = _enc_terms(g, xyz_ref[...], vg_ref[0], wp1f_ref, bp1f_ref,
                     wp2T_ref, bp2_ref, velc_ref, wvel2T_ref, bvel2_ref)
    xk_g = jnp.dot(g[:, 0:64], wkT_ref[...],
                   preferred_element_type=jnp.float32) + bk_ref[...]
    xq = jnp.dot(f_ref[...], wqT_ref[...],
                 preferred_element_type=jnp.float32) + bq_ref[...]
    w = xk_g - xq + enc
    sw = jnp.sum(w, axis=0, keepdims=True)
    sww = jnp.sum(w * w, axis=0, keepdims=True)
    out_ref[0:1, :] += sw
    out_ref[1:2, :] += sww


def _passC_body(g_ref, f_ref, xyz_ref, vg_ref, wqT_ref, bq_ref, wkT_ref,
                bk_ref, wp1f_ref, bp1f_ref, wp2T_ref, bp2_ref, velc_ref,
                wvel2T_ref, bvel2_ref, aw_ref, bw_ref, ww1T_ref, bw1_ref,
                w1_ref, out_ref):
    first = (pl.program_id(0) == 0) & (pl.program_id(1) == 0)

    @pl.when(first)
    def _():
        out_ref[...] = jnp.zeros_like(out_ref)

    g = g_ref[0]
    enc = _enc_terms(g, xyz_ref[...], vg_ref[0], wp1f_ref, bp1f_ref,
                     wp2T_ref, bp2_ref, velc_ref, wvel2T_ref, bvel2_ref)
    xk_g = jnp.dot(g[:, 0:64], wkT_ref[...],
                   preferred_element_type=jnp.float32) + bk_ref[...]
    xq = jnp.dot(f_ref[...], wqT_ref[...],
                 preferred_element_type=jnp.float32) + bq_ref[...]
    w = xk_g - xq + enc
    wb = jax.nn.relu(w * aw_ref[...] + bw_ref[...])
    w1 = jnp.dot(wb, ww1T_ref[...],
                 preferred_element_type=jnp.float32) + bw1_ref[...]
    w1_ref[0] = w1
    out_ref[0:1, :] += jnp.sum(w1, axis=0, keepdims=True)
    out_ref[1:2, :] += jnp.sum(w1 * w1, axis=0, keepdims=True)


def _passD_body(g_ref, w1_ref, f_ref, xyz_ref, vg_ref, wvT_ref, bv_ref,
                wp1f_ref, bp1f_ref, wp2T_ref, bp2_ref, velc_ref, wvel2T_ref,
                bvel2_ref, aw1_ref, bw1_ref, ww2T_ref, bw2_ref,
                y_ref, out_ref, attn_ref):
    first = pl.program_id(0) == 0

    @pl.when(first)
    def _():
        out_ref[...] = jnp.zeros_like(out_ref)

    p = g_ref.shape[1]
    w1 = w1_ref[...].reshape(_NSL * p, 8)
    w1b = jax.nn.relu(w1 * aw1_ref[...] + bw1_ref[...])
    w2 = (jnp.dot(w1b, ww2T_ref[...], preferred_element_type=jnp.float32)
          + bw2_ref[...]).reshape(_NSL, p, 8)
    m = jnp.max(w2, axis=0, keepdims=True)
    e = jnp.exp(w2 - m)
    attn_ref[...] = e / jnp.sum(e, axis=0, keepdims=True)
    xyz_blk = xyz_ref[...]

    def sbody(s, acc):
        g2 = g_ref[s]
        v2 = vg_ref[s]
        enc = _enc_terms(g2, xyz_blk, v2, wp1f_ref, bp1f_ref, wp2T_ref,
                         bp2_ref, velc_ref, wvel2T_ref, bvel2_ref)
        xv_g = jnp.dot(g2[:, 0:64], wvT_ref[...],
                       preferred_element_type=jnp.float32) + bv_ref[...]
        a8 = attn_ref[s]
        a64 = jnp.concatenate([a8] * 8, axis=1)
        return acc + (xv_g + enc) * a64

    out0 = jax.lax.fori_loop(0, _NSL, sbody, jnp.zeros((p, 64), jnp.float32))
    y = out0 + f_ref[...]
    y_ref[...] = y
    out_ref[0:1, :] += jnp.sum(y, axis=0, keepdims=True)
    out_ref[1:2, :] += jnp.sum(y * y, axis=0, keepdims=True)


def _passE_body(y_ref, ar_ref, br2_ref, wrT_ref, br_ref, out_ref):
    yb = jax.nn.relu(y_ref[...] * ar_ref[...] + br2_ref[...])
    out_ref[...] = jnp.dot(yb, wrT_ref[...],
                           preferred_element_type=jnp.float32) + br_ref[...]


def _full(b):
    return pl.BlockSpec(b.shape, lambda *a: tuple(0 for _ in b.shape))


def _gather(src, idx_flat):
    return jnp.take(src, idx_flat, axis=0)


def kernel(xyz, features, offset, velocities, Wq, bq, Wk, bk, Wv, bv, Wp1,
           bp1, gp, betp, Wp2, bp2, Wvel1, bvel1, gvel, betvel, Wvel2, bvel2,
           gw1, bw1g, Ww1, bw1, gw2, bw2g, Ww2, bw2, gr, brg, Wr, br):
    n = xyz.shape[0]
    c = features.shape[1]
    p = _P if n % _P == 0 else n
    nb = n // p
    m = float(n * _NSL)
    eps = 1e-5

    idx = _knn(xyz, payload=False)
    v_g = _knn(velocities, payload=True)
    npad = idx.shape[0]

    idxT = jnp.pad(idx[:n].T, ((0, 0), (0, npad - n)))
    vgT3 = jnp.pad(v_g[:n].T, ((0, 0), (0, npad - n)))[..., None]

    fxz = jnp.concatenate(
        [features, xyz, jnp.zeros((n, 80 - c - 3), jnp.float32)], axis=1)
    g_flat = _gather(fxz, idxT.reshape(-1))
    g3 = g_flat.reshape(_NSL, npad, 80)

    stats = _passA(g3, xyz, vgT3, nb, p)[0]
    s3, sq = stats[0:3], stats[3:6]
    sxy, sxz, syz, sv, svv = stats[6], stats[7], stats[8], stats[9], stats[10]
    mp = s3 / m
    E2 = jnp.stack([
        jnp.stack([sq[0], sxy, sxz]),
        jnp.stack([sxy, sq[1], syz]),
        jnp.stack([sxz, syz, sq[2]]),
    ]) / m
    mean_t = Wp1 @ mp + bp1
    et2 = (jnp.einsum("ci,ij,cj->c", Wp1, E2, Wp1)
           + 2.0 * bp1 * (Wp1 @ mp) + bp1 * bp1)
    var_t = et2 - mean_t * mean_t
    ap = gp / jnp.sqrt(var_t + eps)
    bp_ = betp - mean_t * ap
    wp1f = Wp1.T * ap[None, :]
    bp1f = (bp1 * ap + bp_)[None, :]
    wv0, bv0 = Wvel1[0, 0], bvel1[0]
    mv = sv / m
    mean_tv = wv0 * mv + bv0
    var_tv = wv0 * wv0 * (svv / m - mv * mv)
    av_ = gvel[0] / jnp.sqrt(var_tv + eps)
    bvb = betvel[0] - mean_tv * av_
    velc = jnp.stack([wv0 * av_, bv0 * av_ + bvb]).reshape(1, 2)

    wqT, wkT, wvT, wp2T = Wq.T, Wk.T, Wv.T, Wp2.T
    wvel2T = Wvel2.T
    bq2, bk2, bv2, bp22, bvel22 = (x[None, :] for x in (bq, bk, bv, bp2, bvel2))

    common_in = [g3, features, xyz, vgT3]
    common_specs = [
        pl.BlockSpec((1, p, 80), lambda i, s: (s, i, 0)),
        pl.BlockSpec((p, c), lambda i, s: (i, 0)),
        pl.BlockSpec((p, 3), lambda i, s: (i, 0)),
        pl.BlockSpec((1, p, 1), lambda i, s: (s, i, 0)),
    ]
    wconst = [wqT, bq2, wkT, bk2, wp1f, bp1f, wp2T, bp22, velc, wvel2T,
              bvel22]
    wconst_specs = [_full(x) for x in wconst]

    wstats = pl.pallas_call(
        _passB_body,
        grid=(nb, _NSL),
        in_specs=common_specs + wconst_specs,
        out_specs=pl.BlockSpec((2, c), lambda i, s: (0, 0)),
        out_shape=jax.ShapeDtypeStruct((2, c), jnp.float32),
        compiler_params=pltpu.CompilerParams(
            dimension_semantics=("arbitrary", "arbitrary")),
    )(*common_in, *wconst)
    mean_w = wstats[0] / m
    var_w = wstats[1] / m - mean_w * mean_w
    aw = (gw1 / jnp.sqrt(var_w + eps))[None, :]
    bw_ = bw1g[None, :] - mean_w[None, :] * aw

    ww1T = Ww1.T
    bw12 = bw1[None, :]
    w1_out, w1stats = pl.pallas_call(
        _passC_body,
        grid=(nb, _NSL),
        in_specs=common_specs + wconst_specs
        + [_full(x) for x in (aw, bw_, ww1T, bw12)],
        out_specs=[
            pl.BlockSpec((1, p, 8), lambda i, s: (s, i, 0)),
            pl.BlockSpec((2, 8), lambda i, s: (0, 0)),
        ],
        out_shape=[
            jax.ShapeDtypeStruct((_NSL, npad, 8), jnp.float32),
            jax.ShapeDtypeStruct((2, 8), jnp.float32),
        ],
        compiler_params=pltpu.CompilerParams(
            dimension_semantics=("arbitrary", "arbitrary")),
    )(*common_in, *wconst, aw, bw_, ww1T, bw12)
    mean_w1 = w1stats[0] / m
    var_w1 = w1stats[1] / m - mean_w1 * mean_w1
    aw1 = (gw2 / jnp.sqrt(var_w1 + eps))[None, :]
    bw1_ = bw2g[None, :] - mean_w1[None, :] * aw1

    ww2T = Ww2.T
    bw22 = bw2[None, :]
    dconst = [wvT, bv2, wp1f, bp1f, wp2T, bp22, velc, wvel2T, bvel22,
              aw1, bw1_, ww2T, bw22]
    pd = 200 if n % 200 == 0 else n
    nbd = n // pd
    y_out, ystats = pl.pallas_call(
        _passD_body,
        grid=(nbd,),
        in_specs=[
            pl.BlockSpec((_NSL, pd, 80), lambda i: (0, i, 0)),
            pl.BlockSpec((_NSL, pd, 8), lambda i: (0, i, 0)),
            pl.BlockSpec((pd, c), lambda i: (i, 0)),
            pl.BlockSpec((pd, 3), lambda i: (i, 0)),
            pl.BlockSpec((_NSL, pd, 1), lambda i: (0, i, 0)),
        ] + [_full(x) for x in dconst],
        out_specs=[
            pl.BlockSpec((pd, c), lambda i: (i, 0)),
            pl.BlockSpec((2, c), lambda i: (0, 0)),
        ],
        out_shape=[
            jax.ShapeDtypeStruct((n, c), jnp.float32),
            jax.ShapeDtypeStruct((2, c), jnp.float32),
        ],
        scratch_shapes=[pltpu.VMEM((_NSL, pd, 8), jnp.float32)],
        compiler_params=pltpu.CompilerParams(
            dimension_semantics=("arbitrary",)),
    )(g3, w1_out, features, xyz, vgT3, *dconst)
    mean_y = ystats[0] / n
    var_y = ystats[1] / n - mean_y * mean_y
    ar = (gr / jnp.sqrt(var_y + eps))[None, :]
    br_ = brg[None, :] - mean_y[None, :] * ar

    wrT = Wr.T
    br2 = br[None, :]
    out = pl.pallas_call(
        _passE_body,
        grid=(nb,),
        in_specs=[pl.BlockSpec((p, c), lambda i: (i, 0))]
        + [_full(x) for x in (ar, br_, wrT, br2)],
        out_specs=pl.BlockSpec((p, c), lambda i: (i, 0)),
        out_shape=jax.ShapeDtypeStruct((n, c), jnp.float32),
        compiler_params=pltpu.CompilerParams(
            dimension_semantics=("arbitrary",)),
    )(y_out, ar, br_, wrT, br2)
    return out


# speedup vs baseline: 2.4318x; 1.0739x over previous
import functools

import jax
import jax.numpy as jnp
from jax.experimental import pallas as pl
from jax.experimental.pallas import tpu as pltpu
from jax.experimental.pallas import tpu_sc as plsc

_BIG = 3.0e38
_NSL = 16
_R = 256
_P = 1000


def _knn_body(q_ref, rT_ref, ssq_ref, ssr_ref, out_ref, *, n_chunks, payload):
    lane128 = jax.lax.broadcasted_iota(jnp.int32, (8, 128), 1)
    lane512 = jax.lax.broadcasted_iota(jnp.int32, (8, 512), 1).astype(jnp.float32)
    rTb = rT_ref[...].astype(jnp.bfloat16)

    def slab(r, carry):
        r8 = pl.multiple_of(r * 8, 8)
        q = q_ref[pl.ds(r8, 8), :]
        qq = ssq_ref[pl.ds(r8, 8), :]
        if payload:
            mm = q * rT_ref[...]
        else:
            mm = jnp.dot(q.astype(jnp.bfloat16), rTb,
                         preferred_element_type=jnp.float32)

        def chunk(c, bufs):
            b0, b1, b2, b3, p0, p1, p2, p3 = bufs
            base = c * 128
            d2 = (qq + ssr_ref[0:1, base:base + 128]) \
                - 2.0 * mm[:, base:base + 128]
            if payload:
                pay = jnp.broadcast_to(rT_ref[0:1, base:base + 128], (8, 128))
            else:
                pay = lane128 + c * 128
            m3 = d2 < b3
            nv = jnp.where(m3, d2, b3)
            np_ = jnp.where(m3, pay, p3)
            m2 = nv < b2
            b3n = jnp.where(m2, b2, nv)
            p3n = jnp.where(m2, p2, np_)
            nv2 = jnp.where(m2, nv, b2)
            np2 = jnp.where(m2, np_, p2)
            m1 = nv2 < b1
            b2n = jnp.where(m1, b1, nv2)
            p2n = jnp.where(m1, p1, np2)
            nv1 = jnp.where(m1, nv2, b1)
            np1 = jnp.where(m1, np2, p1)
            m0 = nv1 < b0
            b1n = jnp.where(m0, b0, nv1)
            p1n = jnp.where(m0, p0, np1)
            b0n = jnp.where(m0, nv1, b0)
            p0n = jnp.where(m0, np1, p0)
            return b0n, b1n, b2n, b3n, p0n, p1n, p2n, p3n

        fb = jnp.full((8, 128), _BIG, jnp.float32)
        if payload:
            fp = jnp.zeros((8, 128), jnp.float32)
        else:
            fp = jnp.zeros((8, 128), jnp.int32)
        bufs = (fb, fb, fb, fb, fp, fp, fp, fp)
        for c in range(n_chunks):
            bufs = chunk(c, bufs)
        b0, b1, b2, b3, p0, p1, p2, p3 = bufs
        B = jnp.concatenate([b0, b1, b2, b3], axis=1)
        P = jnp.concatenate([p0, p1, p2, p3], axis=1)
        outs = []
        for _ in range(_NSL):
            rowmin = jnp.min(B, axis=1, keepdims=True)
            eq = B == rowmin
            pos = jnp.min(jnp.where(eq, lane512, 1e9), axis=1, keepdims=True)
            sel = lane512 == pos
            if payload:
                val = jnp.sum(jnp.where(sel, P, 0.0), axis=1, keepdims=True)
            else:
                val = jnp.sum(jnp.where(sel, P, 0), axis=1, keepdims=True)
            outs.append(val)
            B = jnp.where(sel, _BIG, B)
        out_ref[pl.ds(r8, 8), :] = jnp.concatenate(outs, axis=1)
        return carry

    jax.lax.fori_loop(0, _R // 8, slab, 0)


def _knn(coords, payload):
    n, d = coords.shape
    npad = -(-n // _R) * _R
    pad = jnp.full((npad - n, d), 1e6, jnp.float32)
    cpad = jnp.concatenate([coords.astype(jnp.float32), pad], axis=0)
    cT = cpad.T
    ss = jnp.sum(cpad * cpad, axis=1)
    out_dtype = jnp.float32 if payload else jnp.int32
    body = functools.partial(_knn_body, n_chunks=npad // 128, payload=payload)
    return pl.pallas_call(
        body,
        grid=(npad // _R,),
        in_specs=[
            pl.BlockSpec((_R, d), lambda i: (i, 0)),
            pl.BlockSpec((d, npad), lambda i: (0, 0)),
            pl.BlockSpec((_R, 1), lambda i: (i, 0)),
            pl.BlockSpec((1, npad), lambda i: (0, 0)),
        ],
        out_specs=pl.BlockSpec((_R, _NSL), lambda i: (i, 0)),
        out_shape=jax.ShapeDtypeStruct((npad, _NSL), out_dtype),
        compiler_params=pltpu.CompilerParams(
            dimension_semantics=("arbitrary",)),
    )(cpad, cT, ss[:, None], ss[None, :])


def _passA_body(g_ref, xyz_ref, vg_ref, out_ref):
    first = (pl.program_id(0) == 0) & (pl.program_id(1) == 0)

    @pl.when(first)
    def _():
        out_ref[...] = jnp.zeros_like(out_ref)

    g = g_ref[0]
    pr = g[:, 64:67] - xyz_ref[...]
    v = vg_ref[0]
    s3 = jnp.sum(pr, axis=0, keepdims=True)
    sq = jnp.sum(pr * pr, axis=0, keepdims=True)
    sxy = jnp.sum(pr[:, 0:1] * pr[:, 1:2], axis=0, keepdims=True)
    sxz = jnp.sum(pr[:, 0:1] * pr[:, 2:3], axis=0, keepdims=True)
    syz = jnp.sum(pr[:, 1:2] * pr[:, 2:3], axis=0, keepdims=True)
    sv = jnp.sum(v, axis=0, keepdims=True)
    svv = jnp.sum(v * v, axis=0, keepdims=True)
    row = jnp.concatenate(
        [s3, sq, sxy, sxz, syz, sv, svv, jnp.zeros((1, 5), jnp.float32)], axis=1)
    out_ref[...] += row


def _passA(g3, xyz, vg3, nb, p):
    return pl.pallas_call(
        _passA_body,
        grid=(_NSL, nb),
        in_specs=[
            pl.BlockSpec((1, p, 128), lambda s, i: (s, i, 0)),
            pl.BlockSpec((p, 3), lambda s, i: (i, 0)),
            pl.BlockSpec((1, p, 1), lambda s, i: (s, i, 0)),
        ],
        out_specs=pl.BlockSpec((1, 16), lambda s, i: (0, 0)),
        out_shape=jax.ShapeDtypeStruct((1, 16), jnp.float32),
        compiler_params=pltpu.CompilerParams(
            dimension_semantics=("arbitrary", "arbitrary")),
    )(g3, xyz, vg3)


def _enc_terms(g2, xyz_blk, v_blk, wp1f_ref, bp1f_ref, wp2T_ref, bp2_ref,
               velc_ref, wvel2T_ref, bvel2_ref):
    pr = g2[:, 64:67] - xyz_blk
    pe = jax.nn.relu(
        jnp.dot(pr, wp1f_ref[...], preferred_element_type=jnp.float32)
        + bp1f_ref[...])
    p_enc = jnp.dot(pe, wp2T_ref[...], preferred_element_type=jnp.float32) \
        + bp2_ref[...]
    av = velc_ref[0:1, 0:1]
    bv = velc_ref[0:1, 1:2]
    a = jax.nn.relu(v_blk * av + bv)
    v_enc = a * wvel2T_ref[...] + bvel2_ref[...]
    return p_enc + v_enc


def _passB_body(g_ref, f_ref, xyz_ref, vg_ref, wqT_ref, bq_ref, wkT_ref,
                bk_ref, wp1f_ref, bp1f_ref, wp2T_ref, bp2_ref, velc_ref,
                wvel2T_ref, bvel2_ref, out_ref):
    first = (pl.program_id(0) == 0) & (pl.program_id(1) == 0)

    @pl.when(first)
    def _():
        out_ref[...] = jnp.zeros_like(out_ref)

    g = g_ref[0]
    enc = _enc_terms(g, xyz_ref[...], vg_ref[0], wp1f_ref, bp1f_ref,
                     wp2T_ref, bp2_ref, velc_ref, wvel2T_ref, bvel2_ref)
    xk_g = jnp.dot(g[:, 0:64], wkT_ref[...],
                   preferred_element_type=jnp.float32) + bk_ref[...]
    xq = jnp.dot(f_ref[...], wqT_ref[...],
                 preferred_element_type=jnp.float32) + bq_ref[...]
    w = xk_g - xq + enc
    sw = jnp.sum(w, axis=0, keepdims=True)
    sww = jnp.sum(w * w, axis=0, keepdims=True)
    out_ref[0:1, :] += sw
    out_ref[1:2, :] += sww


def _passC_body(g_ref, f_ref, xyz_ref, vg_ref, wqT_ref, bq_ref, wkT_ref,
                bk_ref, wp1f_ref, bp1f_ref, wp2T_ref, bp2_ref, velc_ref,
                wvel2T_ref, bvel2_ref, aw_ref, bw_ref, ww1T_ref, bw1_ref,
                w1_ref, out_ref):
    first = (pl.program_id(0) == 0) & (pl.program_id(1) == 0)

    @pl.when(first)
    def _():
        out_ref[...] = jnp.zeros_like(out_ref)

    g = g_ref[0]
    enc = _enc_terms(g, xyz_ref[...], vg_ref[0], wp1f_ref, bp1f_ref,
                     wp2T_ref, bp2_ref, velc_ref, wvel2T_ref, bvel2_ref)
    xk_g = jnp.dot(g[:, 0:64], wkT_ref[...],
                   preferred_element_type=jnp.float32) + bk_ref[...]
    xq = jnp.dot(f_ref[...], wqT_ref[...],
                 preferred_element_type=jnp.float32) + bq_ref[...]
    w = xk_g - xq + enc
    wb = jax.nn.relu(w * aw_ref[...] + bw_ref[...])
    w1 = jnp.dot(wb, ww1T_ref[...],
                 preferred_element_type=jnp.float32) + bw1_ref[...]
    w1_ref[0] = w1
    out_ref[0:1, :] += jnp.sum(w1, axis=0, keepdims=True)
    out_ref[1:2, :] += jnp.sum(w1 * w1, axis=0, keepdims=True)


def _passD_body(g_ref, w1_ref, f_ref, xyz_ref, vg_ref, wvT_ref, bv_ref,
                wp1f_ref, bp1f_ref, wp2T_ref, bp2_ref, velc_ref, wvel2T_ref,
                bvel2_ref, aw1_ref, bw1_ref, ww2T_ref, bw2_ref,
                y_ref, out_ref, attn_ref):
    first = pl.program_id(0) == 0

    @pl.when(first)
    def _():
        out_ref[...] = jnp.zeros_like(out_ref)

    p = g_ref.shape[1]
    w1 = w1_ref[...].reshape(_NSL * p, 8)
    w1b = jax.nn.relu(w1 * aw1_ref[...] + bw1_ref[...])
    w2 = (jnp.dot(w1b, ww2T_ref[...], preferred_element_type=jnp.float32)
          + bw2_ref[...]).reshape(_NSL, p, 8)
    m = jnp.max(w2, axis=0, keepdims=True)
    e = jnp.exp(w2 - m)
    attn_ref[...] = e / jnp.sum(e, axis=0, keepdims=True)
    xyz_blk = xyz_ref[...]

    def sbody(s, acc):
        g2 = g_ref[s]
        v2 = vg_ref[s]
        enc = _enc_terms(g2, xyz_blk, v2, wp1f_ref, bp1f_ref, wp2T_ref,
                         bp2_ref, velc_ref, wvel2T_ref, bvel2_ref)
        xv_g = jnp.dot(g2[:, 0:64], wvT_ref[...],
                       preferred_element_type=jnp.float32) + bv_ref[...]
        a8 = attn_ref[s]
        a64 = jnp.concatenate([a8] * 8, axis=1)
        return acc + (xv_g + enc) * a64

    out0 = jax.lax.fori_loop(0, _NSL, sbody, jnp.zeros((p, 64), jnp.float32))
    y = out0 + f_ref[...]
    y_ref[...] = y
    out_ref[0:1, :] += jnp.sum(y, axis=0, keepdims=True)
    out_ref[1:2, :] += jnp.sum(y * y, axis=0, keepdims=True)


def _passE_body(y_ref, ar_ref, br2_ref, wrT_ref, br_ref, out_ref):
    yb = jax.nn.relu(y_ref[...] * ar_ref[...] + br2_ref[...])
    out_ref[...] = jnp.dot(yb, wrT_ref[...],
                           preferred_element_type=jnp.float32) + br_ref[...]


def _full(b):
    return pl.BlockSpec(b.shape, lambda *a: tuple(0 for _ in b.shape))


def _sc_gather(src, idx_flat):
    nidx = idx_flat.shape[0]
    vdim = src.shape[1]
    idx2 = idx_flat.reshape(1, nidx)
    mesh = plsc.VectorSubcoreMesh(core_axis_name="core",
                                  subcore_axis_name="subcore")

    @pl.kernel(out_type=jax.ShapeDtypeStruct((nidx, vdim), src.dtype),
               mesh=mesh)
    def k(x_hbm, i_hbm, o_hbm):
        def body(i_vmem, o_vmem):
            pltpu.sync_copy(x_hbm.at[i_vmem.at[0]], o_vmem)

        pltpu.emit_pipeline(
            body,
            grid=(nidx // 128,),
            in_specs=[pl.BlockSpec((1, 128), lambda i: (0, i))],
            out_specs=[pl.BlockSpec((128, vdim), lambda i: (i, 0))],
            core_axis_name="subcore",
            dimension_semantics=(pltpu.PARALLEL,),
        )(i_hbm, o_hbm)

    return k(src, idx2)


def kernel(xyz, features, offset, velocities, Wq, bq, Wk, bk, Wv, bv, Wp1,
           bp1, gp, betp, Wp2, bp2, Wvel1, bvel1, gvel, betvel, Wvel2, bvel2,
           gw1, bw1g, Ww1, bw1, gw2, bw2g, Ww2, bw2, gr, brg, Wr, br):
    n = xyz.shape[0]
    c = features.shape[1]
    p = _P if n % _P == 0 else n
    nb = n // p
    m = float(n * _NSL)
    eps = 1e-5

    idx = _knn(xyz, payload=False)
    v_g = _knn(velocities, payload=True)
    npad = idx.shape[0]

    idxT = jnp.pad(idx[:n].T, ((0, 0), (0, npad - n)))
    vgT3 = jnp.pad(v_g[:n].T, ((0, 0), (0, npad - n)))[..., None]

    fxz = jnp.concatenate(
        [features, xyz, jnp.zeros((n, 128 - c - 3), jnp.float32)], axis=1)
    g_flat = _sc_gather(fxz, idxT.reshape(-1))
    g3 = g_flat.reshape(_NSL, npad, 128)

    stats = _passA(g3, xyz, vgT3, nb, p)[0]
    s3, sq = stats[0:3], stats[3:6]
    sxy, sxz, syz, sv, svv = stats[6], stats[7], stats[8], stats[9], stats[10]
    mp = s3 / m
    E2 = jnp.stack([
        jnp.stack([sq[0], sxy, sxz]),
        jnp.stack([sxy, sq[1], syz]),
        jnp.stack([sxz, syz, sq[2]]),
    ]) / m
    mean_t = Wp1 @ mp + bp1
    et2 = (jnp.einsum("ci,ij,cj->c", Wp1, E2, Wp1)
           + 2.0 * bp1 * (Wp1 @ mp) + bp1 * bp1)
    var_t = et2 - mean_t * mean_t
    ap = gp / jnp.sqrt(var_t + eps)
    bp_ = betp - mean_t * ap
    wp1f = Wp1.T * ap[None, :]
    bp1f = (bp1 * ap + bp_)[None, :]
    wv0, bv0 = Wvel1[0, 0], bvel1[0]
    mv = sv / m
    mean_tv = wv0 * mv + bv0
    var_tv = wv0 * wv0 * (svv / m - mv * mv)
    av_ = gvel[0] / jnp.sqrt(var_tv + eps)
    bvb = betvel[0] - mean_tv * av_
    velc = jnp.stack([wv0 * av_, bv0 * av_ + bvb]).reshape(1, 2)

    wqT, wkT, wvT, wp2T = Wq.T, Wk.T, Wv.T, Wp2.T
    wvel2T = Wvel2.T
    bq2, bk2, bv2, bp22, bvel22 = (x[None, :] for x in (bq, bk, bv, bp2, bvel2))

    common_in = [g3, features, xyz, vgT3]
    common_specs = [
        pl.BlockSpec((1, p, 128), lambda i, s: (s, i, 0)),
        pl.BlockSpec((p, c), lambda i, s: (i, 0)),
        pl.BlockSpec((p, 3), lambda i, s: (i, 0)),
        pl.BlockSpec((1, p, 1), lambda i, s: (s, i, 0)),
    ]
    wconst = [wqT, bq2, wkT, bk2, wp1f, bp1f, wp2T, bp22, velc, wvel2T,
              bvel22]
    wconst_specs = [_full(x) for x in wconst]

    wstats = pl.pallas_call(
        _passB_body,
        grid=(nb, _NSL),
        in_specs=common_specs + wconst_specs,
        out_specs=pl.BlockSpec((2, c), lambda i, s: (0, 0)),
        out_shape=jax.ShapeDtypeStruct((2, c), jnp.float32),
        compiler_params=pltpu.CompilerParams(
            dimension_semantics=("arbitrary", "arbitrary")),
    )(*common_in, *wconst)
    mean_w = wstats[0] / m
    var_w = wstats[1] / m - mean_w * mean_w
    aw = (gw1 / jnp.sqrt(var_w + eps))[None, :]
    bw_ = bw1g[None, :] - mean_w[None, :] * aw

    ww1T = Ww1.T
    bw12 = bw1[None, :]
    w1_out, w1stats = pl.pallas_call(
        _passC_body,
        grid=(nb, _NSL),
        in_specs=common_specs + wconst_specs
        + [_full(x) for x in (aw, bw_, ww1T, bw12)],
        out_specs=[
            pl.BlockSpec((1, p, 8), lambda i, s: (s, i, 0)),
            pl.BlockSpec((2, 8), lambda i, s: (0, 0)),
        ],
        out_shape=[
            jax.ShapeDtypeStruct((_NSL, npad, 8), jnp.float32),
            jax.ShapeDtypeStruct((2, 8), jnp.float32),
        ],
        compiler_params=pltpu.CompilerParams(
            dimension_semantics=("arbitrary", "arbitrary")),
    )(*common_in, *wconst, aw, bw_, ww1T, bw12)
    mean_w1 = w1stats[0] / m
    var_w1 = w1stats[1] / m - mean_w1 * mean_w1
    aw1 = (gw2 / jnp.sqrt(var_w1 + eps))[None, :]
    bw1_ = bw2g[None, :] - mean_w1[None, :] * aw1

    ww2T = Ww2.T
    bw22 = bw2[None, :]
    dconst = [wvT, bv2, wp1f, bp1f, wp2T, bp22, velc, wvel2T, bvel22,
              aw1, bw1_, ww2T, bw22]
    pd = 200 if n % 200 == 0 else n
    nbd = n // pd
    y_out, ystats = pl.pallas_call(
        _passD_body,
        grid=(nbd,),
        in_specs=[
            pl.BlockSpec((_NSL, pd, 128), lambda i: (0, i, 0)),
            pl.BlockSpec((_NSL, pd, 8), lambda i: (0, i, 0)),
            pl.BlockSpec((pd, c), lambda i: (i, 0)),
            pl.BlockSpec((pd, 3), lambda i: (i, 0)),
            pl.BlockSpec((_NSL, pd, 1), lambda i: (0, i, 0)),
        ] + [_full(x) for x in dconst],
        out_specs=[
            pl.BlockSpec((pd, c), lambda i: (i, 0)),
            pl.BlockSpec((2, c), lambda i: (0, 0)),
        ],
        out_shape=[
            jax.ShapeDtypeStruct((n, c), jnp.float32),
            jax.ShapeDtypeStruct((2, c), jnp.float32),
        ],
        scratch_shapes=[pltpu.VMEM((_NSL, pd, 8), jnp.float32)],
        compiler_params=pltpu.CompilerParams(
            dimension_semantics=("arbitrary",)),
    )(g3, w1_out, features, xyz, vgT3, *dconst)
    mean_y = ystats[0] / n
    var_y = ystats[1] / n - mean_y * mean_y
    ar = (gr / jnp.sqrt(var_y + eps))[None, :]
    br_ = brg[None, :] - mean_y[None, :] * ar

    wrT = Wr.T
    br2 = br[None, :]
    out = pl.pallas_call(
        _passE_body,
        grid=(nb,),
        in_specs=[pl.BlockSpec((p, c), lambda i: (i, 0))]
        + [_full(x) for x in (ar, br_, wrT, br2)],
        out_specs=pl.BlockSpec((p, c), lambda i: (i, 0)),
        out_shape=jax.ShapeDtypeStruct((n, c), jnp.float32),
        compiler_params=pltpu.CompilerParams(
            dimension_semantics=("arbitrary",)),
    )(y_out, ar, br_, wrT, br2)
    return out
